# baseline (speedup 1.0000x reference)
import functools

import jax
import jax.numpy as jnp
from jax.experimental import pallas as pl
from jax.experimental.pallas import tpu as pltpu

HIDDEN = 16
LANES = 128
NUM_CLASSES = 7
OUT_LANES = 8
NEG_BIG = -1e30


def _round_up(x, m):
    return ((x + m - 1) // m) * m


def _slab_offsets(f_pad):
    o_w1 = 0
    o_b1 = _round_up(max(f_pad, 8), 8)
    o_w2 = o_b1 + 8
    o_b2 = o_w2 + LANES
    return o_w1, o_b1, o_w2, o_b2


def _gcn_kernel(x_hbm, a_hbm, slab_ref, out_ref,
                x_buf, a_buf, xw_buf, hw_buf, x_sems, a_sems,
                *, num_features, f_pad, n_pad, tm, tx, tl):
    n_chunks = n_pad // tm
    _, o_b1, o_w2, o_b2 = _slab_offsets(f_pad)
    w1 = slab_ref[0:num_features, :].astype(jnp.bfloat16)
    b1 = slab_ref[o_b1:o_b1 + 1, :]
    w2 = slab_ref[o_w2:o_w2 + LANES, :].astype(jnp.bfloat16)
    b2 = slab_ref[o_b2:o_b2 + 1, :]

    n_x = n_pad // tx

    def x_copy(k, buf_slot):
        return pltpu.make_async_copy(
            x_hbm.at[pl.ds(k * tx, tx), :], x_buf.at[buf_slot], x_sems.at[buf_slot])

    def a_copy(i):
        return pltpu.make_async_copy(
            a_hbm.at[pl.ds(i * tm, tm), :],
            a_buf.at[pl.ds(i * tm, tm), :], a_sems.at[i])

    x_copy(0, 0).start()
    if n_x > 1:
        x_copy(1, 1).start()

    for k in range(n_x):
        x_copy(k, k % 2).wait()
        xb = x_buf[k % 2].astype(jnp.bfloat16)
        xw_buf[pl.ds(k * tx, tx), :] = jnp.dot(
            xb, w1, preferred_element_type=jnp.float32).astype(jnp.bfloat16)
        if k + 2 < n_x:
            x_copy(k + 2, k % 2).start()
        if k == n_x - 1:
            for i in range(n_chunks):
                a_copy(i).start()

    xw = xw_buf[...]
    for i in range(n_chunks):
        a_copy(i).wait()
        pre1 = jnp.dot(a_buf[pl.ds(i * tm, tm), :], xw,
                       preferred_element_type=jnp.float32)
        h1 = jnp.maximum(pre1 + b1, 0.0).astype(jnp.bfloat16)
        hw_buf[pl.ds(i * tm, tm), :] = jnp.dot(
            h1, w2, preferred_element_type=jnp.float32).astype(jnp.bfloat16)

    hw = hw_buf[...]
    for i in range(n_pad // tl):
        logits = jnp.dot(a_buf[pl.ds(i * tl, tl), :], hw,
                         preferred_element_type=jnp.float32) + b2
        m = jnp.max(logits, axis=1, keepdims=True)
        z = logits - m
        lse = jnp.log(jnp.sum(jnp.exp(z), axis=1, keepdims=True))
        out_ref[pl.ds(i * tl, tl), :] = (z - lse)[:, :OUT_LANES]


@jax.jit
def _forward(x, a_hat_pad, slab):
    num_nodes, num_features = x.shape
    n_pad = a_hat_pad.shape[0]
    f_pad = _round_up(max(num_features, 1), LANES)
    tm = min(1024, n_pad)
    tx = min(1024, n_pad)
    tl = min(2048, n_pad)

    cparams = pltpu.CompilerParams(
        dimension_semantics=("arbitrary",),
        vmem_limit_bytes=60 * 1024 * 1024,
    )
    out = pl.pallas_call(
        functools.partial(_gcn_kernel, num_features=num_features,
                          f_pad=f_pad, n_pad=n_pad, tm=tm, tx=tx, tl=tl),
        grid=(1,),
        in_specs=[
            pl.BlockSpec(memory_space=pl.ANY),
            pl.BlockSpec(memory_space=pl.ANY),
            pl.BlockSpec(slab.shape, lambda i: (0, 0)),
        ],
        out_specs=pl.BlockSpec((n_pad, OUT_LANES), lambda i: (0, 0)),
        out_shape=jax.ShapeDtypeStruct((n_pad, OUT_LANES), jnp.float32),
        scratch_shapes=[
            pltpu.VMEM((2, tx, num_features), jnp.float32),
            pltpu.VMEM((n_pad, n_pad), jnp.bfloat16),
            pltpu.VMEM((n_pad, LANES), jnp.bfloat16),
            pltpu.VMEM((n_pad, LANES), jnp.bfloat16),
            pltpu.SemaphoreType.DMA((2,)),
            pltpu.SemaphoreType.DMA((n_pad // tm,)),
        ],
        compiler_params=cparams,
    )(x, a_hat_pad, slab)

    return out[:num_nodes, :NUM_CLASSES]


def kernel(x, a_hat_pad, slab):
    return _forward(x, a_hat_pad, slab)


# speedup vs baseline: 1.0946x; 1.0946x over previous
import functools

import jax
import jax.numpy as jnp
from jax.experimental import pallas as pl
from jax.experimental.pallas import tpu as pltpu

HIDDEN = 16
LANES = 128
NUM_CLASSES = 7
OUT_LANES = 7


def _round_up(x, m):
    return ((x + m - 1) // m) * m


def _slab_offsets(f_pad):
    o_w1 = 0
    o_b1 = _round_up(max(f_pad, 8), 8)
    o_w2 = o_b1 + 8
    o_b2 = o_w2 + LANES
    return o_w1, o_b1, o_w2, o_b2


def _xw_kernel(x_ref, slab_ref, xw_ref, *, num_features):
    w1 = slab_ref[0:num_features, :].astype(jnp.bfloat16)
    xb = x_ref[...].astype(jnp.bfloat16)
    xw_ref[...] = jnp.dot(
        xb, w1, preferred_element_type=jnp.float32).astype(jnp.bfloat16)


def _layer1_kernel(a_ref, xw_ref, slab_ref, hw_ref, *, f_pad):
    _, o_b1, o_w2, _ = _slab_offsets(f_pad)
    b1 = slab_ref[o_b1:o_b1 + 1, :]
    w2 = slab_ref[o_w2:o_w2 + LANES, :].astype(jnp.bfloat16)
    pre1 = jnp.dot(a_ref[...], xw_ref[...],
                   preferred_element_type=jnp.float32)
    h1 = jnp.maximum(pre1 + b1, 0.0).astype(jnp.bfloat16)
    hw_ref[...] = jnp.dot(h1, w2,
                          preferred_element_type=jnp.float32).astype(jnp.bfloat16)


def _layer2_kernel(a_ref, hw_ref, slab_ref, out_ref, *, f_pad):
    _, _, _, o_b2 = _slab_offsets(f_pad)
    b2 = slab_ref[o_b2:o_b2 + 1, :]
    logits = jnp.dot(a_ref[...], hw_ref[...],
                     preferred_element_type=jnp.float32) + b2
    m = jnp.max(logits, axis=1, keepdims=True)
    z = logits - m
    lse = jnp.log(jnp.sum(jnp.exp(z), axis=1, keepdims=True))
    out_ref[...] = (z - lse)[:, :OUT_LANES]


@jax.jit
def _forward(x, a_hat_pad, slab):
    num_nodes, num_features = x.shape
    n_pad = a_hat_pad.shape[0]
    f_pad = _round_up(max(num_features, 1), LANES)

    vmem_limit = 64 * 1024 * 1024 * 3 // 4
    cparams = pltpu.CompilerParams(
        dimension_semantics=("parallel",),
        vmem_limit_bytes=vmem_limit,
    )
    slab_spec = pl.BlockSpec(slab.shape, lambda i: (0, 0))
    cparams_seq = pltpu.CompilerParams(
        dimension_semantics=("arbitrary",),
        vmem_limit_bytes=vmem_limit,
    )

    tmx = 2048
    xw = pl.pallas_call(
        functools.partial(_xw_kernel, num_features=num_features),
        grid=(n_pad // tmx,),
        in_specs=[
            pl.BlockSpec((tmx, num_features), lambda i: (i, 0)),
            slab_spec,
        ],
        out_specs=pl.BlockSpec((tmx, LANES), lambda i: (i, 0)),
        out_shape=jax.ShapeDtypeStruct((n_pad, LANES), jnp.bfloat16),
        compiler_params=cparams_seq,
    )(x, slab)

    tm = 1024
    grid = (n_pad // tm,)
    a_spec = pl.BlockSpec((tm, n_pad), lambda i: (i, 0))
    skinny_spec = pl.BlockSpec((n_pad, LANES), lambda i: (0, 0))
    hw_out_spec = pl.BlockSpec((tm, LANES), lambda i: (i, 0))

    hw = pl.pallas_call(
        functools.partial(_layer1_kernel, f_pad=f_pad),
        grid=grid,
        in_specs=[a_spec, skinny_spec, slab_spec],
        out_specs=hw_out_spec,
        out_shape=jax.ShapeDtypeStruct((n_pad, LANES), jnp.bfloat16),
        compiler_params=cparams,
    )(a_hat_pad, xw, slab)

    out = pl.pallas_call(
        functools.partial(_layer2_kernel, f_pad=f_pad),
        grid=grid,
        in_specs=[a_spec, skinny_spec, slab_spec],
        out_specs=pl.BlockSpec((tm, OUT_LANES), lambda i: (i, 0)),
        out_shape=jax.ShapeDtypeStruct((n_pad, OUT_LANES), jnp.float32),
        compiler_params=cparams,
    )(a_hat_pad, hw, slab)

    return out


def kernel(x, a_hat_pad, slab):
    return _forward(x, a_hat_pad, slab)


# speedup vs baseline: 1.1015x; 1.0063x over previous
import functools

import jax
import jax.numpy as jnp
from jax.experimental import pallas as pl
from jax.experimental.pallas import tpu as pltpu

HIDDEN = 16
LANES = 128
NUM_CLASSES = 7
OUT_LANES = 7


def _round_up(x, m):
    return ((x + m - 1) // m) * m


def _slab_offsets(f_pad):
    o_w1 = 0
    o_b1 = _round_up(max(f_pad, 8), 8)
    o_w2 = o_b1 + 8
    o_b2 = o_w2 + LANES
    return o_w1, o_b1, o_w2, o_b2


def _xw_kernel(x_ref, slab_ref, xw_ref, *, num_features):
    w1 = slab_ref[0:num_features, :].astype(jnp.bfloat16)
    xb = x_ref[...].astype(jnp.bfloat16)
    xw_ref[...] = jnp.dot(
        xb, w1, preferred_element_type=jnp.float32).astype(jnp.bfloat16)


def _layer1_kernel(a_ref, xw_ref, slab_ref, hw_ref, *, f_pad):
    _, o_b1, o_w2, _ = _slab_offsets(f_pad)
    b1 = slab_ref[o_b1:o_b1 + 1, :]
    w2 = slab_ref[o_w2:o_w2 + LANES, :].astype(jnp.bfloat16)
    pre1 = jnp.dot(a_ref[...], xw_ref[...],
                   preferred_element_type=jnp.float32)
    h1 = jnp.maximum(pre1 + b1, 0.0).astype(jnp.bfloat16)
    hw_ref[...] = jnp.dot(h1, w2,
                          preferred_element_type=jnp.float32).astype(jnp.bfloat16)


def _layer2_kernel(a_ref, hw_ref, slab_ref, out_ref, *, f_pad):
    _, _, _, o_b2 = _slab_offsets(f_pad)
    b2 = slab_ref[o_b2:o_b2 + 1, :]
    logits = jnp.dot(a_ref[...], hw_ref[...],
                     preferred_element_type=jnp.float32) + b2
    m = jnp.max(logits, axis=1, keepdims=True)
    z = logits - m
    lse = jnp.log(jnp.sum(jnp.exp(z), axis=1, keepdims=True))
    out_ref[...] = (z - lse)[:, :OUT_LANES]


@jax.jit
def _forward(x, a_hat_pad, slab):
    num_nodes, num_features = x.shape
    n_pad = a_hat_pad.shape[0]
    f_pad = _round_up(max(num_features, 1), LANES)

    vmem_limit = 64 * 1024 * 1024 * 3 // 4
    cparams = pltpu.CompilerParams(
        dimension_semantics=("parallel",),
        vmem_limit_bytes=vmem_limit,
    )
    slab_spec = pl.BlockSpec(slab.shape, lambda i: (0, 0))
    cparams_seq = pltpu.CompilerParams(
        dimension_semantics=("arbitrary",),
        vmem_limit_bytes=vmem_limit,
    )

    tmx = 1024
    xw = pl.pallas_call(
        functools.partial(_xw_kernel, num_features=num_features),
        grid=(n_pad // tmx,),
        in_specs=[
            pl.BlockSpec((tmx, num_features), lambda i: (i, 0)),
            slab_spec,
        ],
        out_specs=pl.BlockSpec((tmx, LANES), lambda i: (i, 0)),
        out_shape=jax.ShapeDtypeStruct((n_pad, LANES), jnp.bfloat16),
        compiler_params=cparams_seq,
    )(x, slab)

    tm = 1024
    grid = (n_pad // tm,)
    a_spec = pl.BlockSpec((tm, n_pad), lambda i: (i, 0))
    skinny_spec = pl.BlockSpec((n_pad, LANES), lambda i: (0, 0))
    hw_out_spec = pl.BlockSpec((tm, LANES), lambda i: (i, 0))

    hw = pl.pallas_call(
        functools.partial(_layer1_kernel, f_pad=f_pad),
        grid=grid,
        in_specs=[a_spec, skinny_spec, slab_spec],
        out_specs=hw_out_spec,
        out_shape=jax.ShapeDtypeStruct((n_pad, LANES), jnp.bfloat16),
        compiler_params=cparams,
    )(a_hat_pad, xw, slab)

    out = pl.pallas_call(
        functools.partial(_layer2_kernel, f_pad=f_pad),
        grid=grid,
        in_specs=[a_spec, skinny_spec, slab_spec],
        out_specs=pl.BlockSpec((tm, OUT_LANES), lambda i: (i, 0)),
        out_shape=jax.ShapeDtypeStruct((n_pad, OUT_LANES), jnp.float32),
        compiler_params=cparams,
    )(a_hat_pad, hw, slab)

    return out


def kernel(x, a_hat_pad, slab):
    return _forward(x, a_hat_pad, slab)
